# Initial kernel scaffold; baseline (speedup 1.0000x reference)
import jax
import jax.numpy as jnp
from jax.experimental import pallas as pl


def kernel(feats, patch_ids, num_patches):
    raise NotImplementedError("write your pallas kernel here")


# speedup vs baseline: 1.4036x; 1.4036x over previous
import functools

import jax
import jax.numpy as jnp
from jax import lax
from jax.experimental import pallas as pl
from jax.experimental.pallas import tpu as pltpu
from jax.experimental.pallas import tpu_sc as plsc

N_FEATS = 4
C = 96
HW = 384 * 384
NUM_PATCHES = 4096

NC = 2
NS = 16
NW = NC * NS
B_PER_W = NUM_PATCHES // NW


def _sc_gather(feats_flat, pid):
    mesh = plsc.VectorSubcoreMesh(
        core_axis_name="c", subcore_axis_name="s", num_cores=NC,
        num_subcores=NS)

    @functools.partial(
        pl.kernel,
        out_type=jax.ShapeDtypeStruct((N_FEATS, C, NUM_PATCHES), jnp.float32),
        mesh=mesh,
        scratch_types=[
            pltpu.VMEM((B_PER_W,), jnp.int32),
            pltpu.VMEM((C, B_PER_W), jnp.int32),
            pltpu.VMEM((C, B_PER_W), jnp.float32),
            pltpu.SemaphoreType.DMA,
        ],
    )
    def k(feats_hbm, pid_hbm, out_hbm, pid_v, idx_v, buf_v, sem):
        wid = lax.axis_index("s") * NC + lax.axis_index("c")
        base = wid * B_PER_W
        for f in range(N_FEATS):
            pltpu.sync_copy(pid_hbm.at[f, pl.ds(base, B_PER_W)], pid_v)

            def build(c, carry):
                off = (f * C + c) * HW
                for i in range(B_PER_W // 16):
                    v = pid_v[pl.ds(i * 16, 16)]
                    idx_v[c, pl.ds(i * 16, 16)] = v + off
                return carry

            lax.fori_loop(0, C, build, 0, unroll=False)

            def fire(c, carry):
                pltpu.async_copy(
                    feats_hbm.at[idx_v.at[c]], buf_v.at[c], sem)
                return carry

            lax.fori_loop(0, C, fire, 0, unroll=False)
            pltpu.make_async_copy(
                out_hbm.at[f, :, pl.ds(base, B_PER_W)], buf_v, sem).wait()
            pltpu.sync_copy(buf_v, out_hbm.at[f, :, pl.ds(base, B_PER_W)])

    return k(feats_flat, pid)


def _tc_normalize(xt):
    def body(x_ref, o_ref):
        x = x_ref[...]
        ss = jnp.sum(x * x, axis=0, keepdims=True)
        y = x / (jnp.sqrt(ss) + 1e-7)
        o_ref[...] = y.T

    return pl.pallas_call(
        body,
        grid=(N_FEATS,),
        in_specs=[pl.BlockSpec((None, C, NUM_PATCHES), lambda i: (i, 0, 0))],
        out_specs=pl.BlockSpec((None, NUM_PATCHES, C), lambda i: (i, 0, 0)),
        out_shape=jax.ShapeDtypeStruct((N_FEATS, NUM_PATCHES, C),
                                       jnp.float32),
    )(xt)


def kernel(feats, patch_ids, num_patches):
    del num_patches
    feats_flat = feats.reshape(-1)
    pid = patch_ids.reshape(N_FEATS, NUM_PATCHES).astype(jnp.int32)
    xt = _sc_gather(feats_flat, pid)
    return _tc_normalize(xt)


# speedup vs baseline: 4.0768x; 2.9045x over previous
import functools

import jax
import jax.numpy as jnp
from jax import lax
from jax.experimental import pallas as pl
from jax.experimental.pallas import tpu as pltpu
from jax.experimental.pallas import tpu_sc as plsc

N_FEATS = 4
C = 96
H = 384
W = 384
HW = H * W
NUM_PATCHES = 4096

NC = 2
NS = 16
NW = NC * NS
B_PER_W = NUM_PATCHES // NW


def _sc_gather(feats_flat, pid):
    mesh = plsc.VectorSubcoreMesh(
        core_axis_name="c", subcore_axis_name="s", num_cores=NC,
        num_subcores=NS)

    @functools.partial(
        pl.kernel,
        out_type=jax.ShapeDtypeStruct((N_FEATS, C, NUM_PATCHES), jnp.float32),
        mesh=mesh,
        scratch_types=[
            pltpu.VMEM((B_PER_W,), jnp.int32),
            pltpu.VMEM((B_PER_W,), jnp.int32),
            pltpu.VMEM((C, B_PER_W), jnp.int32),
            pltpu.VMEM((C, B_PER_W), jnp.float32),
            pltpu.SemaphoreType.DMA,
        ],
    )
    def k(feats_hbm, pid_hbm, out_hbm, pid_v, toff_v, idx_v, buf_v, sem):
        wid = lax.axis_index("s") * NC + lax.axis_index("c")
        base = wid * B_PER_W
        for f in range(N_FEATS):
            pltpu.sync_copy(pid_hbm.at[f, pl.ds(base, B_PER_W)], pid_v)

            for i in range(B_PER_W // 16):
                p = pid_v[pl.ds(i * 16, 16)]
                t = lax.shift_right_logical(p, 7)
                h = lax.shift_right_logical(t * 21846, 16)
                tw = t - 3 * h
                ti = lax.shift_right_logical(h, 3) * 3 + tw
                off = (lax.shift_left(ti, 10)
                       + lax.shift_left(h & 7, 7)
                       + (p & 127))
                toff_v[pl.ds(i * 16, 16)] = off

            def build(c, carry):
                pb = (f * C + c) * HW
                for i in range(B_PER_W // 16):
                    v = toff_v[pl.ds(i * 16, 16)]
                    idx_v[c, pl.ds(i * 16, 16)] = v + pb
                return carry

            lax.fori_loop(0, C, build, 0, unroll=False)

            def fire(c, carry):
                pltpu.async_copy(
                    feats_hbm.at[idx_v.at[c]], buf_v.at[c], sem)
                return carry

            lax.fori_loop(0, C, fire, 0, unroll=False)
            pltpu.make_async_copy(
                out_hbm.at[f, :, pl.ds(base, B_PER_W)], buf_v, sem).wait()
            pltpu.sync_copy(buf_v, out_hbm.at[f, :, pl.ds(base, B_PER_W)])

    return k(feats_flat, pid)


def _tc_normalize(xt):
    def body(x_ref, o_ref):
        x = x_ref[...]
        ss = jnp.sum(x * x, axis=0, keepdims=True)
        y = x / (jnp.sqrt(ss) + 1e-7)
        o_ref[...] = y.T

    return pl.pallas_call(
        body,
        grid=(N_FEATS,),
        in_specs=[pl.BlockSpec((None, C, NUM_PATCHES), lambda i: (i, 0, 0))],
        out_specs=pl.BlockSpec((None, NUM_PATCHES, C), lambda i: (i, 0, 0)),
        out_shape=jax.ShapeDtypeStruct((N_FEATS, NUM_PATCHES, C),
                                       jnp.float32),
    )(xt)


def kernel(feats, patch_ids, num_patches):
    del num_patches
    feats_flat = (feats.reshape(N_FEATS, C, H // 8, 8, W // 128, 128)
                  .transpose(0, 1, 2, 4, 3, 5)
                  .reshape(-1))
    pid = patch_ids.reshape(N_FEATS, NUM_PATCHES).astype(jnp.int32)
    xt = _sc_gather(feats_flat, pid)
    return _tc_normalize(xt)


# speedup vs baseline: 4.7213x; 1.1581x over previous
import functools

import jax
import jax.numpy as jnp
from jax import lax
from jax.experimental import pallas as pl
from jax.experimental.pallas import tpu as pltpu
from jax.experimental.pallas import tpu_sc as plsc

N_FEATS = 4
C = 96
H = 384
W = 384
HW = H * W
NUM_PATCHES = 4096

NC = 2
NS = 16
NW = NC * NS
B_PER_W = NUM_PATCHES // NW
CT = C // 8


def _sc_gather(feats_flat, pid):
    mesh = plsc.VectorSubcoreMesh(
        core_axis_name="c", subcore_axis_name="s", num_cores=NC,
        num_subcores=NS)

    @functools.partial(
        pl.kernel,
        out_type=jax.ShapeDtypeStruct((N_FEATS, CT, NW, 8, 128),
                                      jnp.float32),
        mesh=mesh,
        scratch_types=[
            pltpu.VMEM((N_FEATS, B_PER_W), jnp.int32),
            pltpu.VMEM((B_PER_W,), jnp.int32),
            pltpu.VMEM((C, B_PER_W), jnp.int32),
            pltpu.VMEM((2, CT, 8, B_PER_W), jnp.float32),
            pltpu.SemaphoreType.DMA,
            pltpu.SemaphoreType.DMA,
        ],
    )
    def k(feats_hbm, pid_hbm, out_hbm, pid_v, toff_v, idx_v, buf_v,
          gsem, wsem):
        wid = lax.axis_index("s") * NC + lax.axis_index("c")
        base = wid * B_PER_W
        pltpu.sync_copy(pid_hbm.at[:, 0, pl.ds(base, B_PER_W)], pid_v)
        for f in range(N_FEATS):
            buf = buf_v.at[f % 2]
            if f >= 2:
                pltpu.make_async_copy(
                    buf, out_hbm.at[f - 2, :, wid], wsem).wait()

            for i in range(B_PER_W // 16):
                p = pid_v[f, pl.ds(i * 16, 16)]
                t = lax.shift_right_logical(p, 7)
                h = lax.shift_right_logical(t * 21846, 16)
                tw = t - 3 * h
                ti = lax.shift_right_logical(h, 3) * 3 + tw
                off = (lax.shift_left(ti, 10)
                       + lax.shift_left(h & 7, 7)
                       + (p & 127))
                toff_v[pl.ds(i * 16, 16)] = off

            def build_fire(c, carry):
                pb = (f * C + c) * HW
                for i in range(B_PER_W // 16):
                    v = toff_v[pl.ds(i * 16, 16)]
                    idx_v[c, pl.ds(i * 16, 16)] = v + pb
                ct = lax.shift_right_logical(c, 3)
                cs = c & 7
                pltpu.async_copy(
                    feats_hbm.at[idx_v.at[c]], buf.at[ct, cs], gsem)
                return carry

            lax.fori_loop(0, C, build_fire, 0, unroll=False)
            pltpu.make_async_copy(
                out_hbm.at[f, :, wid], buf, gsem).wait()
            pltpu.async_copy(buf, out_hbm.at[f, :, wid], wsem)
        for f in range(N_FEATS - 2, N_FEATS):
            pltpu.make_async_copy(
                buf_v.at[f % 2], out_hbm.at[f, :, wid], wsem).wait()

    return k(feats_flat, pid)


def _tc_normalize(xt):
    def body(x_ref, o_ref):
        x = x_ref[...]
        ss = jnp.sum(x * x, axis=0, keepdims=True)
        y = x / (jnp.sqrt(ss) + 1e-7)
        o_ref[...] = y.T

    return pl.pallas_call(
        body,
        grid=(N_FEATS,),
        in_specs=[pl.BlockSpec((None, C, NUM_PATCHES), lambda i: (i, 0, 0))],
        out_specs=pl.BlockSpec((None, NUM_PATCHES, C), lambda i: (i, 0, 0)),
        out_shape=jax.ShapeDtypeStruct((N_FEATS, NUM_PATCHES, C),
                                       jnp.float32),
    )(xt)


def kernel(feats, patch_ids, num_patches):
    del num_patches
    feats_flat = (feats.reshape(N_FEATS, C, H // 8, 8, W // 128, 128)
                  .transpose(0, 1, 2, 4, 3, 5)
                  .reshape(-1))
    pid = patch_ids.astype(jnp.int32)
    raw = _sc_gather(feats_flat, pid)
    xt = raw.transpose(0, 1, 3, 2, 4).reshape(N_FEATS, C, NUM_PATCHES)
    return _tc_normalize(xt)
